# baseline (speedup 1.0000x reference)
import functools
import jax
import jax.numpy as jnp
from jax import lax
from jax.experimental import pallas as pl
from jax.experimental.pallas import tpu as pltpu
from jax.experimental.pallas import tpu_sc as plsc

BATCH, SEQ, GSIZE = 4096, 50, 8
N = BATCH * SEQ
D_SP, D_AB, D_IT, D_MV = 32, 16, 16, 16
D_OUT = D_SP + D_AB + D_IT + D_MV + 4

_info = plsc.get_sparse_core_info()
NC, NS, L = _info.num_cores, _info.num_subcores, _info.num_lanes
NW = NC * NS
PER_W = N // NW
B = 128
CHUNKS = PER_W // B

_DIMS = (D_SP, D_AB, D_IT, D_MV)
_OFFS = (0, D_SP, D_SP + D_AB, D_SP + D_AB + D_IT)


def _make_kernel():
    mesh = plsc.VectorSubcoreMesh(core_axis_name="c", subcore_axis_name="s")

    scratch = []
    for _ in range(2):
        scratch.append(pltpu.VMEM((B, GSIZE), jnp.float32))
        scratch.extend(pltpu.VMEM((B,), jnp.int32) for _ in range(4))
        scratch.extend(pltpu.VMEM((B, d), jnp.float32) for d in _DIMS)
        scratch.append(pltpu.VMEM((B, D_OUT), jnp.float32))
    scratch.extend(pltpu.SemaphoreType.DMA for _ in range(6))

    @functools.partial(
        pl.kernel,
        mesh=mesh,
        out_type=jax.ShapeDtypeStruct((N, D_OUT), jnp.float32),
        compiler_params=pltpu.CompilerParams(
            needs_layout_passes=False, use_tc_tiling_on_sc=False),
        scratch_types=scratch,
    )
    def k(x_hbm, sp_hbm, ab_hbm, it_hbm, mv_hbm, out_hbm, *s):
        x_v = (s[0], s[10])
        idx = (s[1:5], s[11:15])
        gat = (s[5:9], s[15:19])
        out_v = (s[9], s[19])
        xsem, gsem, osem = (s[20], s[21]), (s[22], s[23]), (s[24], s[25])
        tables = (sp_hbm, ab_hbm, it_hbm, mv_hbm)

        wid = lax.axis_index("s") * NC + lax.axis_index("c")
        lane = lax.iota(jnp.int32, L)
        rq = lax.shift_right_logical(lane, 2)
        cq = lax.bitwise_and(lane, 3)

        def rows_of(j):
            return pl.ds(wid * PER_W + j * B, B)

        def fire_x(j, p):
            pltpu.async_copy(x_hbm.at[rows_of(j)], x_v[p], xsem[p])

        def wait_x(p):
            pltpu.make_async_copy(x_hbm.at[pl.ds(0, B)], x_v[p], xsem[p]).wait()

        def fire_gathers(p):
            for t in range(4):
                pltpu.async_copy(tables[t].at[idx[p][t]], gat[p][t], gsem[p])

        def wait_gathers(p):
            for t in range(4):
                pltpu.make_async_copy(
                    tables[t].at[pl.ds(0, B)], gat[p][t], gsem[p]).wait()

        def fire_out(j, p):
            pltpu.async_copy(out_v[p], out_hbm.at[rows_of(j)], osem[p])

        def wait_out(p):
            pltpu.make_async_copy(out_v[p], out_hbm.at[pl.ds(0, B)], osem[p]).wait()

        def extract_ids_and_tails(p):
            for kk in range(B // L):
                rows = lane + kk * L
                for t in range(4):
                    cols = jnp.full((L,), t, jnp.int32)
                    vals = plsc.load_gather(x_v[p], [rows, cols])
                    idx[p][t][pl.ds(kk * L, L)] = jnp.maximum(vals.astype(jnp.int32), 0)
            for q in range(B // 4):
                trows = rq + 4 * q
                tail = plsc.load_gather(x_v[p], [trows, cq + 4])
                plsc.store_scatter(out_v[p], [trows, cq + (D_OUT - 4)], tail)

        def assemble(p):
            for r in range(B):
                for t in range(4):
                    for h in range(_DIMS[t] // L):
                        out_v[p][r, pl.ds(_OFFS[t] + h * L, L)] = \
                            gat[p][t][r, pl.ds(h * L, L)]

        def body(j, p):
            wait_x(p)

            @pl.when(j >= 2)
            def _():
                wait_out(p)

            extract_ids_and_tails(p)
            fire_gathers(p)

            @pl.when(j + 1 < CHUNKS)
            def _():
                fire_x(j + 1, 1 - p)

            @pl.when(j >= 1)
            def _():
                wait_gathers(1 - p)
                assemble(1 - p)
                fire_out(j - 1, 1 - p)

        fire_x(0, 0)

        def pair(g, c):
            body(2 * g, 0)
            body(2 * g + 1, 1)
            return c

        lax.fori_loop(0, CHUNKS // 2, pair, 0)

        wait_gathers(1)
        assemble(1)
        fire_out(CHUNKS - 1, 1)
        wait_out(0)
        wait_out(1)

    return k


_sc_lookup = _make_kernel()


def kernel(x, species_emb, ability_emb, item_emb, move_emb, group_idx):
    x2 = x.reshape(N, GSIZE)
    out = _sc_lookup(x2, species_emb, ability_emb, item_emb, move_emb)
    return out.reshape(BATCH, SEQ, D_OUT)
